# baseline (speedup 1.0000x reference)
import functools

import jax
import jax.numpy as jnp
from jax import lax
from jax.experimental import pallas as pl
from jax.experimental.pallas import tpu as pltpu
from jax.experimental.pallas import tpu_sc as plsc

_B, _N, _D = 16, 4096, 256
_K = 10
_LANES = 16
_NCHUNK = _N // _LANES
_NC = 2


def _sc_body(mask_hbm, x_hbm, out_hbm, mask_v, idx_v, rows_v, sem):
    wid = lax.axis_index("s") * _NC + lax.axis_index("c")

    @pl.when(wid < _B)
    def _():
        b = wid
        pltpu.sync_copy(mask_hbm.at[b], mask_v)
        idx_v[...] = jnp.full((_LANES,), b * _N, jnp.int32)

        def step(t, carry):
            j = _NCHUNK - 1 - t
            m = mask_v[pl.ds(j * _LANES, _LANES)]
            cs = plsc.cumsum(m)
            total = jnp.sum(m)
            rc = total - cs + m
            c = rc + carry
            keep = (m > 0) & (c <= _K)
            slot = jnp.clip(_K - c, 0, _LANES - 1)
            gidx = b * _N + j * _LANES + lax.iota(jnp.int32, _LANES)
            plsc.store_scatter(idx_v, [slot], gidx, mask=keep)
            return carry + total

        lax.fori_loop(0, _NCHUNK, step, jnp.int32(0))
        pltpu.async_copy(x_hbm.at[idx_v], rows_v, sem).wait()
        pltpu.sync_copy(rows_v.at[pl.ds(0, _K)], out_hbm.at[b])


def kernel(x, layer_layouts, node_mask):
    del layer_layouts
    mask_i32 = node_mask.astype(jnp.int32)
    x2d = x.reshape(_B * _N, _D)

    run = functools.partial(
        pl.kernel,
        out_type=jax.ShapeDtypeStruct((_B, _K, _D), jnp.float32),
        mesh=plsc.VectorSubcoreMesh(core_axis_name="c", subcore_axis_name="s"),
        compiler_params=pltpu.CompilerParams(
            use_tc_tiling_on_sc=False, needs_layout_passes=False
        ),
        scratch_types=[
            pltpu.VMEM((_N,), jnp.int32),
            pltpu.VMEM((_LANES,), jnp.int32),
            pltpu.VMEM((_LANES, _D), jnp.float32),
            pltpu.SemaphoreType.DMA,
        ],
    )(_sc_body)

    out = run(mask_i32, x2d)
    return out.reshape(_B, _K * _D)


# speedup vs baseline: 1.0589x; 1.0589x over previous
import functools

import jax
import jax.numpy as jnp
from jax import lax
from jax.experimental import pallas as pl
from jax.experimental.pallas import tpu as pltpu
from jax.experimental.pallas import tpu_sc as plsc

_B, _N, _D = 16, 4096, 256
_K = 10
_LANES = 16
_NCHUNK = _N // _LANES
_NC = 2


def _sc_body(mask_hbm, x_hbm, out_hbm, mask_v, idx_v, rows_v, sem):
    wid = lax.axis_index("s") * _NC + lax.axis_index("c")

    @pl.when(wid < _B)
    def _():
        b = wid
        pltpu.sync_copy(mask_hbm.at[b], mask_v)
        idx_v[...] = jnp.full((_LANES,), b * _N, jnp.int32)

        def cond(state):
            j, carry = state
            return (carry < _K) & (j >= 0)

        def step(state):
            j, carry = state
            m = mask_v[pl.ds(j * _LANES, _LANES)]
            cs = plsc.cumsum(m)
            total = jnp.sum(m)
            rc = total - cs + m
            c = rc + carry
            keep = (m > 0) & (c <= _K)
            slot = jnp.clip(_K - c, 0, _LANES - 1)
            gidx = b * _N + j * _LANES + lax.iota(jnp.int32, _LANES)
            plsc.store_scatter(idx_v, [slot], gidx, mask=keep)
            return j - 1, carry + total

        lax.while_loop(cond, step, (jnp.int32(_NCHUNK - 1), jnp.int32(0)))
        pltpu.async_copy(x_hbm.at[idx_v], rows_v, sem).wait()
        pltpu.sync_copy(rows_v.at[pl.ds(0, _K)], out_hbm.at[b])


def kernel(x, layer_layouts, node_mask):
    del layer_layouts
    mask_i32 = node_mask.astype(jnp.int32)
    x2d = x.reshape(_B * _N, _D)

    run = functools.partial(
        pl.kernel,
        out_type=jax.ShapeDtypeStruct((_B, _K, _D), jnp.float32),
        mesh=plsc.VectorSubcoreMesh(core_axis_name="c", subcore_axis_name="s"),
        compiler_params=pltpu.CompilerParams(
            use_tc_tiling_on_sc=False, needs_layout_passes=False
        ),
        scratch_types=[
            pltpu.VMEM((_N,), jnp.int32),
            pltpu.VMEM((_LANES,), jnp.int32),
            pltpu.VMEM((_LANES, _D), jnp.float32),
            pltpu.SemaphoreType.DMA,
        ],
    )(_sc_body)

    out = run(mask_i32, x2d)
    return out.reshape(_B, _K * _D)


# speedup vs baseline: 3.4043x; 3.2151x over previous
import functools

import jax
import jax.numpy as jnp
from jax import lax
from jax.experimental import pallas as pl
from jax.experimental.pallas import tpu as pltpu
from jax.experimental.pallas import tpu_sc as plsc

_B, _N, _D = 16, 4096, 256
_K = 10
_LANES = 16
_NCHUNK = _N // _LANES
_NC = 2


def _sc_body(mask_hbm, x_hbm, out_hbm, mask_v, idx_v, rows_v, sem):
    wid = lax.axis_index("s") * _NC + lax.axis_index("c")

    @pl.when(wid < _B)
    def _():
        b = wid
        pltpu.sync_copy(mask_hbm.at[b], mask_v)
        idx_v[...] = jnp.full((_LANES,), b * _N, jnp.int32)

        def cond(state):
            j, carry = state
            return (carry < _K) & (j >= 0)

        def step(state):
            j, carry = state
            m = mask_v[pl.ds(j * _LANES, _LANES)]
            cs = plsc.cumsum(m)
            total = jnp.sum(m)
            rc = total - cs + m
            c = rc + carry
            keep = (m > 0) & (c <= _K)
            slot = jnp.clip(_K - c, 0, _LANES - 1)
            gidx = b * _N + j * _LANES + lax.iota(jnp.int32, _LANES)
            plsc.store_scatter(idx_v, [slot], gidx, mask=keep)
            return j - 1, carry + total

        lax.while_loop(cond, step, (jnp.int32(_NCHUNK - 1), jnp.int32(0)))
        pltpu.async_copy(x_hbm.at[idx_v], rows_v, sem).wait()
        pltpu.sync_copy(rows_v, out_hbm.at[b])


def kernel(x, layer_layouts, node_mask):
    del layer_layouts
    mask_i32 = node_mask.astype(jnp.int32)
    x2d = x.reshape(_B * _N, _D)

    run = functools.partial(
        pl.kernel,
        out_type=jax.ShapeDtypeStruct((_B, _LANES, _D), jnp.float32),
        mesh=plsc.VectorSubcoreMesh(core_axis_name="c", subcore_axis_name="s"),
        compiler_params=pltpu.CompilerParams(
            use_tc_tiling_on_sc=True, needs_layout_passes=False
        ),
        scratch_types=[
            pltpu.VMEM((_N,), jnp.int32),
            pltpu.VMEM((_LANES,), jnp.int32),
            pltpu.VMEM((_LANES, _D), jnp.float32),
            pltpu.SemaphoreType.DMA,
        ],
    )(_sc_body)

    out = run(mask_i32, x2d)
    return out[:, :_K, :].reshape(_B, _K * _D)


# speedup vs baseline: 5.7832x; 1.6988x over previous
import jax
import jax.numpy as jnp
from jax.experimental import pallas as pl
from jax.experimental.pallas import tpu as pltpu

_B, _N, _D = 16, 4096, 256
_K = 10


def _idx_body(mask_ref, idx_ref):
    m = mask_ref[...].astype(jnp.int32)
    v = jax.lax.broadcasted_iota(jnp.int32, (_B, _N), 1) * m
    cols = []
    for _ in range(_K):
        mx = jnp.max(v, axis=1)
        cols.append(jnp.maximum(mx, 0))
        v = jnp.where(v == mx[:, None], -1, v)
    cols.reverse()
    idx_ref[...] = jnp.stack(cols, axis=1)


def _gather_body(idx_ref, *refs):
    o_ref = refs[-1]
    c = pl.program_id(0)
    rows = []
    for i in range(_B):
        r = idx_ref[i, c] % 8
        rows.append(refs[i][pl.ds(r, 1), :])
    o_ref[...] = jnp.concatenate(rows, axis=0)


def _x_spec(i):
    return pl.BlockSpec(
        (8, _D), lambda c, idx_ref, i=i: ((i * _N + idx_ref[i, c]) // 8, 0)
    )


def kernel(x, layer_layouts, node_mask):
    del layer_layouts

    idx = pl.pallas_call(
        _idx_body,
        out_shape=jax.ShapeDtypeStruct((_B, _K), jnp.int32),
    )(node_mask)

    x2d = x.reshape(_B * _N, _D)
    out = pl.pallas_call(
        _gather_body,
        grid_spec=pltpu.PrefetchScalarGridSpec(
            num_scalar_prefetch=1,
            grid=(_K,),
            in_specs=[_x_spec(i) for i in range(_B)],
            out_specs=pl.BlockSpec((_B, _D), lambda c, idx_ref: (0, c)),
        ),
        out_shape=jax.ShapeDtypeStruct((_B, _K * _D), jnp.float32),
    )(idx, *([x2d] * _B))
    return out


# speedup vs baseline: 11.6593x; 2.0161x over previous
import jax
import jax.numpy as jnp
from jax import lax
from jax.experimental import pallas as pl
from jax.experimental.pallas import tpu as pltpu

_B, _N, _D = 16, 4096, 256
_K = 10
_T = 128


def _topk_rounds(v, base):
    cols = []
    for _ in range(_K):
        mx = jnp.max(v, axis=1)
        cols.append(jnp.maximum(mx + base, 0))
        v = jnp.where(v == mx[:, None], -1, v)
    cols.reverse()
    return jnp.stack(cols, axis=1)


_W = 24


def _body(mask_ref, x_hbm, o_ref, win_v, sem):
    m = mask_ref[...]
    mt = m[:, _N - _T:]
    tail_cnt = jnp.sum(mt, axis=1)
    all_dense = jnp.min(tail_cnt) >= _K

    def tail_path(_):
        vt = lax.broadcasted_iota(jnp.int32, (_B, _T), 1) * mt
        return _topk_rounds(vt, _N - _T)

    def full_path(_):
        v = lax.broadcasted_iota(jnp.int32, (_B, _N), 1) * m
        return _topk_rounds(v, 0)

    slots = lax.cond(all_dense, tail_path, full_path, 0)

    bi = lax.broadcasted_iota(jnp.int32, (_B, _K), 0)
    ci = lax.broadcasted_iota(jnp.int32, (_B, _K), 1)
    contiguous = jnp.sum(
        jnp.where(slots == slots[:, :1] + ci, 1, 0)
    ) == _B * _K

    @pl.when(contiguous)
    def _():
        copies, offs = [], []
        for i in range(_B):
            base = jnp.sum(jnp.where(bi == i, slots * (ci == 0), 0))
            w = jnp.minimum((base // 8) * 8, _N - _W)
            offs.append(base - w)
            copies.append(
                pltpu.make_async_copy(
                    x_hbm.at[pl.ds(i * _N + w, _W), :], win_v.at[i], sem
                )
            )
        for cp in copies:
            cp.start()
        for cp in copies:
            cp.wait()
        sels = [
            pltpu.roll(win_v[i], (_W - offs[i]) % _W, 0)[:_K]
            for i in range(_B)
        ]
        for c in range(_K):
            o_ref[:, pl.ds(c * _D, _D)] = jnp.concatenate(
                [sels[i][c : c + 1, :] for i in range(_B)], axis=0
            )

    @pl.when(jnp.logical_not(contiguous))
    def _():
        bi2 = lax.broadcasted_iota(jnp.int32, (_B, _K * _D), 0)
        cd2 = lax.broadcasted_iota(jnp.int32, (_B, _K * _D), 1) // _D

        def one(g, acc):
            i = g // _K
            c = g % _K
            s = jnp.sum(jnp.where((bi == i) & (ci == c), slots, 0))
            w = (s // 8) * 8
            cp = pltpu.make_async_copy(
                x_hbm.at[pl.ds(i * _N + w, 8), :],
                win_v.at[0, pl.ds(0, 8)],
                sem,
            )
            cp.start()
            cp.wait()
            row = pltpu.roll(win_v[0, :8, :], (8 - (s - w)) % 8, 0)[:1]
            rowt = jnp.concatenate([row] * _K, axis=1)
            return jnp.where((bi2 == i) & (cd2 == c), rowt, acc)

        acc = lax.fori_loop(
            0, _B * _K, one, jnp.zeros((_B, _K * _D), jnp.float32)
        )
        o_ref[...] = acc


def kernel(x, layer_layouts, node_mask):
    del layer_layouts
    x2d = x.reshape(_B * _N, _D)
    mask_i32 = node_mask.astype(jnp.int32)

    out = pl.pallas_call(
        _body,
        in_specs=[
            pl.BlockSpec((_B, _N), lambda: (0, 0)),
            pl.BlockSpec(memory_space=pl.ANY),
        ],
        out_specs=pl.BlockSpec((_B, _K * _D), lambda: (0, 0)),
        out_shape=jax.ShapeDtypeStruct((_B, _K * _D), jnp.float32),
        scratch_shapes=[
            pltpu.VMEM((_B, _W, _D), jnp.float32),
            pltpu.SemaphoreType.DMA,
        ],
    )(mask_i32, x2d)
    return out


# speedup vs baseline: 12.3455x; 1.0589x over previous
import jax
import jax.numpy as jnp
from jax import lax
from jax.experimental import pallas as pl
from jax.experimental.pallas import tpu as pltpu

_B, _N, _D = 16, 4096, 256
_K = 10
_T = 128
_W = 24


def _topk_rounds(v, base):
    cols = []
    for _ in range(_K):
        mx = jnp.max(v, axis=1)
        cols.append(jnp.maximum(mx + base, 0))
        v = jnp.where(v == mx[:, None], -1, v)
    cols.reverse()
    return jnp.stack(cols, axis=1)


def _body(mask_ref, x_hbm, o_ref, win_v, sem):
    mt = mask_ref[:, _N - _T:].astype(jnp.int32)
    all_dense = jnp.min(jnp.sum(mt, axis=1)) >= _K

    def tail_path(_):
        vt = lax.broadcasted_iota(jnp.int32, (_B, _T), 1) * mt
        return _topk_rounds(vt, _N - _T)

    def full_path(_):
        m = mask_ref[...].astype(jnp.int32)
        v = lax.broadcasted_iota(jnp.int32, (_B, _N), 1) * m
        return _topk_rounds(v, 0)

    slots = lax.cond(all_dense, tail_path, full_path, 0)

    bi = lax.broadcasted_iota(jnp.int32, (_B, _K), 0)
    ci = lax.broadcasted_iota(jnp.int32, (_B, _K), 1)
    base_v = slots[:, :1]
    base = jnp.max(base_v)
    uniform = (
        jnp.sum(jnp.where(slots == base_v + ci, 1, 0)) == _B * _K
    ) & (jnp.min(base_v) == base)

    @pl.when(uniform)
    def _():
        w = jnp.minimum((base // 8) * 8, _N - _W)
        pltpu.make_async_copy(
            x_hbm.at[:, pl.ds(w, _W), :], win_v, sem
        ).start()
        pltpu.make_async_copy(
            x_hbm.at[:, pl.ds(w, _W), :], win_v, sem
        ).wait()
        rolled = pltpu.roll(win_v[...], (_W - (base - w)) % _W, 1)
        for c in range(_K):
            o_ref[:, pl.ds(c * _D, _D)] = rolled[:, c, :]

    @pl.when(jnp.logical_not(uniform))
    def _():
        bi2 = lax.broadcasted_iota(jnp.int32, (_B, _K * _D), 0)
        cd2 = lax.broadcasted_iota(jnp.int32, (_B, _K * _D), 1) // _D

        def one(g, acc):
            i = g // _K
            c = g % _K
            s = jnp.sum(jnp.where((bi == i) & (ci == c), slots, 0))
            w = (s // 8) * 8
            cp = pltpu.make_async_copy(
                x_hbm.at[pl.ds(i, 1), pl.ds(w, 8), :],
                win_v.at[pl.ds(0, 1), pl.ds(0, 8)],
                sem,
            )
            cp.start()
            cp.wait()
            row = pltpu.roll(win_v[0, :8, :], (8 - (s - w)) % 8, 0)[:1]
            rowt = jnp.concatenate([row] * _K, axis=1)
            return jnp.where((bi2 == i) & (cd2 == c), rowt, acc)

        acc = lax.fori_loop(
            0, _B * _K, one, jnp.zeros((_B, _K * _D), jnp.float32)
        )
        o_ref[...] = acc


def kernel(x, layer_layouts, node_mask):
    del layer_layouts
    mask_u8 = node_mask.view(jnp.int8)

    out = pl.pallas_call(
        _body,
        in_specs=[
            pl.BlockSpec((_B, _N), lambda: (0, 0)),
            pl.BlockSpec(memory_space=pl.ANY),
        ],
        out_specs=pl.BlockSpec((_B, _K * _D), lambda: (0, 0)),
        out_shape=jax.ShapeDtypeStruct((_B, _K * _D), jnp.float32),
        scratch_shapes=[
            pltpu.VMEM((_B, _W, _D), jnp.float32),
            pltpu.SemaphoreType.DMA,
        ],
    )(mask_u8, x)
    return out


# speedup vs baseline: 15.2753x; 1.2373x over previous
import jax
import jax.numpy as jnp
from jax import lax
from jax.experimental import pallas as pl
from jax.experimental.pallas import tpu as pltpu

_B, _N, _D = 16, 4096, 256
_K = 10
_T = 128
_W = 24


def _topk_rounds(v, base):
    cols = []
    for _ in range(_K):
        mx = jnp.max(v, axis=1)
        cols.append(jnp.maximum(mx + base, 0))
        v = jnp.where(v == mx[:, None], -1, v)
    cols.reverse()
    return jnp.stack(cols, axis=1)


def _body(tail_ref, x_hbm, mask_hbm, o_ref, win_v, mfull_v, sem, sem2):
    wspec = _N - _W
    spec_cp = pltpu.make_async_copy(
        x_hbm.at[:, pl.ds(wspec, _W), :], win_v, sem
    )
    spec_cp.start()

    mt = tail_ref[...].astype(jnp.int32)
    all_dense = jnp.min(jnp.sum(mt, axis=1)) >= _K

    @pl.when(jnp.logical_not(all_dense))
    def _():
        cp = pltpu.make_async_copy(mask_hbm, mfull_v, sem2)
        cp.start()
        cp.wait()

    def tail_path(_):
        vt = lax.broadcasted_iota(jnp.int32, (_B, _T), 1) * mt
        return _topk_rounds(vt, _N - _T)

    def full_path(_):
        m = mfull_v[...].astype(jnp.int32)
        v = lax.broadcasted_iota(jnp.int32, (_B, _N), 1) * m
        return _topk_rounds(v, 0)

    slots = lax.cond(all_dense, tail_path, full_path, 0)

    bi = lax.broadcasted_iota(jnp.int32, (_B, _K), 0)
    ci = lax.broadcasted_iota(jnp.int32, (_B, _K), 1)
    base_v = slots[:, :1]
    base = jnp.max(base_v)
    fast = (
        (jnp.sum(jnp.where(slots == base_v + ci, 1, 0)) == _B * _K)
        & (jnp.min(base_v) == base)
        & (base >= wspec)
    )
    spec_cp.wait()

    @pl.when(fast)
    def _():
        rolled = pltpu.roll(win_v[...], (_W - (base - wspec)) % _W, 1)
        for c in range(_K):
            o_ref[:, pl.ds(c * _D, _D)] = rolled[:, c, :]

    @pl.when(jnp.logical_not(fast))
    def _():
        bi2 = lax.broadcasted_iota(jnp.int32, (_B, _K * _D), 0)
        cd2 = lax.broadcasted_iota(jnp.int32, (_B, _K * _D), 1) // _D

        def one(g, acc):
            i = g // _K
            c = g % _K
            s = jnp.sum(jnp.where((bi == i) & (ci == c), slots, 0))
            w = (s // 8) * 8
            cp = pltpu.make_async_copy(
                x_hbm.at[pl.ds(i, 1), pl.ds(w, 8), :],
                win_v.at[pl.ds(0, 1), pl.ds(0, 8)],
                sem,
            )
            cp.start()
            cp.wait()
            row = pltpu.roll(win_v[0, :8, :], (8 - (s - w)) % 8, 0)[:1]
            rowt = jnp.concatenate([row] * _K, axis=1)
            return jnp.where((bi2 == i) & (cd2 == c), rowt, acc)

        acc = lax.fori_loop(
            0, _B * _K, one, jnp.zeros((_B, _K * _D), jnp.float32)
        )
        o_ref[...] = acc


def kernel(x, layer_layouts, node_mask):
    del layer_layouts
    mask_i8 = node_mask.view(jnp.int8)

    out = pl.pallas_call(
        _body,
        grid=(1,),
        in_specs=[
            pl.BlockSpec((_B, _T), lambda i: (0, (_N // _T) - 1)),
            pl.BlockSpec(memory_space=pl.ANY),
            pl.BlockSpec(memory_space=pl.ANY),
        ],
        out_specs=pl.BlockSpec((_B, _K * _D), lambda i: (0, 0)),
        out_shape=jax.ShapeDtypeStruct((_B, _K * _D), jnp.float32),
        scratch_shapes=[
            pltpu.VMEM((_B, _W, _D), jnp.float32),
            pltpu.VMEM((_B, _N), jnp.int8),
            pltpu.SemaphoreType.DMA,
            pltpu.SemaphoreType.DMA,
        ],
    )(mask_i8, x, mask_i8)
    return out
